# baseline (speedup 1.0000x reference)
import functools

import jax
import jax.numpy as jnp
from jax import lax
from jax.experimental import pallas as pl
from jax.experimental.pallas import tpu as pltpu
from jax.experimental.pallas import tpu_sc as plsc

_EMBED = 64
_LANES = 16
_NC = 2
_NS = 16
_NW = _NC * _NS
_CHUNK = 128
_NBUF = 4


@functools.lru_cache(maxsize=None)
def _make_idx_transpose(nb: int, nl: int):
    def body(x_ref, o_ref):
        o_ref[...] = jnp.swapaxes(x_ref[...], 0, 1).reshape(nl, nb // _CHUNK, _CHUNK)

    return pl.pallas_call(
        body,
        out_shape=jax.ShapeDtypeStruct((nl, nb // _CHUNK, _CHUNK), jnp.int32),
    )


@functools.lru_cache(maxsize=None)
def _make_emb_kernel(nl: int, nb: int):
    assert nb == _NW * _CHUNK and nl % _NBUF == 0
    npairs = _CHUNK // 2
    mesh = plsc.VectorSubcoreMesh(core_axis_name="c", subcore_axis_name="s")

    @functools.partial(
        pl.kernel,
        out_type=jax.ShapeDtypeStruct((nb * nl // 2, _CHUNK), jnp.float32),
        mesh=mesh,
        scratch_types=[
            pltpu.VMEM((nl, 1, _CHUNK), jnp.int32),
            pltpu.VMEM((_NBUF, 1, _CHUNK), jnp.int32),
            pltpu.VMEM((_NBUF, _CHUNK, _CHUNK), jnp.float32),
            pltpu.VMEM((_NBUF, npairs, _CHUNK), jnp.float32),
            pltpu.SemaphoreType.DMA((_NBUF,)),
            pltpu.SemaphoreType.DMA((_NBUF,)),
        ],
        compiler_params=pltpu.CompilerParams(
            use_tc_tiling_on_sc=True, needs_layout_passes=False
        ),
    )
    def emb(idx_hbm, table_hbm, out_hbm, idx_v, idxh_v, rows_v, srows_v,
            gsem, ssem):
        wid = lax.axis_index("s") * _NC + lax.axis_index("c")
        obase = wid * (nl * npairs)
        pltpu.sync_copy(idx_hbm.at[:, pl.ds(wid, 1)], idx_v)

        def gather_issue(l, b):
            for j in range(_CHUNK // _LANES):
                sl = pl.ds(j * _LANES, _LANES)
                idxh_v[b, 0, sl] = idx_v[l, 0, sl] >> 1
            pltpu.async_copy(
                table_hbm.at[idxh_v.at[b, 0]], rows_v.at[b], gsem.at[b]
            )

        def gather_wait(b):
            pltpu.make_async_copy(
                table_hbm.at[pl.ds(0, _CHUNK)], rows_v.at[b], gsem.at[b]
            ).wait()

        def scatter_issue(l, b):
            pltpu.async_copy(
                srows_v.at[b],
                out_hbm.at[pl.ds(obase + l * npairs, npairs)],
                ssem.at[b],
            )

        def scatter_wait(b):
            pltpu.make_async_copy(
                srows_v.at[b],
                out_hbm.at[pl.ds(0, npairs)],
                ssem.at[b],
            ).wait()

        iota16 = jnp.arange(_LANES, dtype=jnp.int32)

        def select_scale(l, b):
            @pl.loop(0, _CHUNK // _LANES)
            def _(k):
                i0 = k * _LANES
                par16 = (idx_v[l, 0, pl.ds(i0, _LANES)] & 1) * _EMBED
                row16 = i0 + iota16
                srow16 = (i0 >> 1) + (iota16 >> 1)
                scol16 = (iota16 & 1) * _EMBED
                for e in range(_EMBED):
                    v16 = plsc.load_gather(rows_v.at[b], [row16, par16 + e])
                    plsc.store_scatter(
                        srows_v.at[b], [srow16, scol16 + e], v16 * 8.0
                    )

        def step(l, b, first, last):
            gather_wait(b)
            if not first:
                scatter_wait(b)
            select_scale(l, b)
            if not last:
                gather_issue(l + _NBUF, b)
            scatter_issue(l, b)

        for b in range(_NBUF):
            gather_issue(b, b)
        for b in range(_NBUF):
            step(b, b, first=True, last=False)
        @pl.loop(1, nl // _NBUF - 1)
        def _(g):
            l0 = g * _NBUF
            for b in range(_NBUF):
                step(l0 + b, b, first=False, last=False)
        for b in range(_NBUF):
            step(nl - _NBUF + b, b, first=False, last=True)
        for b in range(_NBUF):
            scatter_wait(b)

    return emb


def kernel(x, input_embedding_table):
    nb, nl = x.shape
    nv = input_embedding_table.shape[0]
    idx3 = _make_idx_transpose(nb, nl)(x)
    table2 = input_embedding_table.reshape(nv // 2, 2 * _EMBED)
    out2 = _make_emb_kernel(nl, nb)(idx3, table2)
    return (
        out2.reshape(_NW, nl, _CHUNK // 2, 2, _EMBED)
        .transpose(0, 2, 3, 1, 4)
        .reshape(nb, nl, _EMBED)
    )


# speedup vs baseline: 2.8580x; 2.8580x over previous
import functools

import jax
import jax.numpy as jnp
from jax import lax
from jax.experimental import pallas as pl
from jax.experimental.pallas import tpu as pltpu
from jax.experimental.pallas import tpu_sc as plsc

_EMBED = 64
_LANES = 16
_NC = 2
_NS = 16
_NW = _NC * _NS
_CHUNK = 128
_NBUF = 8
_LEAD = 6


@functools.lru_cache(maxsize=None)
def _make_emb_kernel(ntok: int):
    npw = ntok // _NW
    nchunk = npw // _CHUNK
    assert nchunk % _NBUF == 0 and nchunk // _NBUF >= 3
    mesh = plsc.VectorSubcoreMesh(core_axis_name="c", subcore_axis_name="s")

    @functools.partial(
        pl.kernel,
        out_type=jax.ShapeDtypeStruct((ntok, _EMBED), jnp.float32),
        mesh=mesh,
        scratch_types=[
            pltpu.VMEM((npw,), jnp.int32),
            pltpu.VMEM((_NBUF, _CHUNK, _EMBED), jnp.float32),
            pltpu.SemaphoreType.DMA((_NBUF,)),
            pltpu.SemaphoreType.DMA((_NBUF,)),
        ],
        compiler_params=pltpu.CompilerParams(use_tc_tiling_on_sc=False),
    )
    def emb(idx_hbm, table_hbm, out_hbm, idx_v, rows_v, gsem, ssem):
        wid = lax.axis_index("s") * _NC + lax.axis_index("c")
        base = wid * npw
        pltpu.sync_copy(idx_hbm.at[pl.ds(base, npw)], idx_v)

        def gather_issue(k, b):
            pltpu.async_copy(
                table_hbm.at[idx_v.at[pl.ds(k * _CHUNK, _CHUNK)]],
                rows_v.at[b],
                gsem.at[b],
            )

        def gather_wait(b):
            pltpu.make_async_copy(
                table_hbm.at[pl.ds(0, _CHUNK)], rows_v.at[b], gsem.at[b]
            ).wait()

        def scatter_issue(k, b):
            pltpu.async_copy(
                rows_v.at[b], out_hbm.at[pl.ds(base + k * _CHUNK, _CHUNK)], ssem.at[b]
            )

        def scatter_wait(b):
            pltpu.make_async_copy(
                rows_v.at[b], out_hbm.at[pl.ds(base, _CHUNK)], ssem.at[b]
            ).wait()

        def scale(b):
            @pl.loop(0, _CHUNK, unroll=8)
            def _(i):
                for j in range(_EMBED // _LANES):
                    sl = pl.ds(j * _LANES, _LANES)
                    rows_v[b, i, sl] = rows_v[b, i, sl] * 8.0

        for g in range(_LEAD):
            gather_issue(g, g)

        for g in range(_NBUF):
            b = g
            gather_wait(b)
            scale(b)
            scatter_issue(g, b)
            if g >= 2:
                scatter_wait((g - 2) % _NBUF)
            gather_issue(g + _LEAD, (g + _LEAD) % _NBUF)

        @pl.loop(1, nchunk // _NBUF - 1)
        def _(s):
            k0 = s * _NBUF
            for b in range(_NBUF):
                k = k0 + b
                gather_wait(b)
                scale(b)
                scatter_issue(k, b)
                scatter_wait((b + _LEAD) % _NBUF)
                gather_issue(k + _LEAD, (b + _LEAD) % _NBUF)

        for g in range(nchunk - _NBUF, nchunk):
            b = g % _NBUF
            gather_wait(b)
            scale(b)
            scatter_issue(g, b)
            if g + _LEAD < nchunk:
                scatter_wait((b + _LEAD) % _NBUF)
                gather_issue(g + _LEAD, (b + _LEAD) % _NBUF)

        for b in range(_NBUF):
            scatter_wait(b)

    return emb


def kernel(x, input_embedding_table):
    b, l = x.shape
    ntok = b * l
    idx = x.reshape(ntok)
    out = _make_emb_kernel(ntok)(idx, input_embedding_table)
    return out.reshape(b, l, _EMBED)


# speedup vs baseline: 3.2220x; 1.1274x over previous
import functools

import jax
import jax.numpy as jnp
from jax import lax
from jax.experimental import pallas as pl
from jax.experimental.pallas import tpu as pltpu
from jax.experimental.pallas import tpu_sc as plsc

_EMBED = 64
_LANES = 16
_NC = 2
_NS = 16
_NW = _NC * _NS
_CHUNK = 128
_NBUF = 8
_LEAD = 6


@functools.lru_cache(maxsize=None)
def _make_emb_kernel(nl: int, nb: int):
    nlt = nl // _NBUF
    assert nb == _NW * _CHUNK and nl % _NBUF == 0 and nlt >= 3
    mesh = plsc.VectorSubcoreMesh(core_axis_name="c", subcore_axis_name="s")

    @functools.partial(
        pl.kernel,
        out_type=jax.ShapeDtypeStruct((nb, nl * _EMBED), jnp.float32),
        mesh=mesh,
        scratch_types=[
            pltpu.VMEM((nlt, 1, _NBUF, _CHUNK), jnp.int32),
            pltpu.VMEM((_NBUF, _CHUNK, _EMBED), jnp.float32),
            pltpu.SemaphoreType.DMA((_NBUF,)),
            pltpu.SemaphoreType.DMA((_NBUF,)),
        ],
        compiler_params=pltpu.CompilerParams(use_tc_tiling_on_sc=False),
    )
    def emb(idx_hbm, table_hbm, out_hbm, idx_v, rows_v, gsem, ssem):
        wid = lax.axis_index("s") * _NC + lax.axis_index("c")
        col = wid * _CHUNK
        pltpu.sync_copy(idx_hbm.at[:, pl.ds(wid, 1)], idx_v)

        def gather_issue(lt, ls, b):
            pltpu.async_copy(
                table_hbm.at[idx_v.at[lt, 0, ls]], rows_v.at[b], gsem.at[b]
            )

        def gather_wait(b):
            pltpu.make_async_copy(
                table_hbm.at[pl.ds(0, _CHUNK)], rows_v.at[b], gsem.at[b]
            ).wait()

        def scatter_issue(lt, ls, b):
            l = lt * _NBUF + ls
            pltpu.async_copy(
                rows_v.at[b],
                out_hbm.at[pl.ds(col, _CHUNK), pl.ds(l * _EMBED, _EMBED)],
                ssem.at[b],
            )

        def scatter_wait(b):
            pltpu.make_async_copy(
                rows_v.at[b],
                out_hbm.at[pl.ds(0, _CHUNK), pl.ds(0, _EMBED)],
                ssem.at[b],
            ).wait()

        def scale(b):
            @pl.loop(0, _CHUNK, unroll=8)
            def _(i):
                for j in range(_EMBED // _LANES):
                    sl = pl.ds(j * _LANES, _LANES)
                    rows_v[b, i, sl] = rows_v[b, i, sl] * 8.0

        for ls in range(_LEAD):
            gather_issue(0, ls, ls)

        for ls in range(_NBUF):
            gather_wait(ls)
            scale(ls)
            scatter_issue(0, ls, ls)
            if ls >= 2:
                scatter_wait((ls - 2) % _NBUF)
            gather_issue((ls + _LEAD) // _NBUF, (ls + _LEAD) % _NBUF,
                         (ls + _LEAD) % _NBUF)

        @pl.loop(1, nlt - 1)
        def _(lt):
            for ls in range(_NBUF):
                gather_wait(ls)
                scale(ls)
                scatter_issue(lt, ls, ls)
                scatter_wait((ls + _LEAD) % _NBUF)
                gather_issue(lt + (ls + _LEAD) // _NBUF, (ls + _LEAD) % _NBUF,
                             (ls + _LEAD) % _NBUF)

        for ls in range(_NBUF):
            gather_wait(ls)
            scale(ls)
            scatter_issue(nlt - 1, ls, ls)
            if ls + _LEAD < _NBUF:
                scatter_wait(ls + _LEAD)
                gather_issue(nlt - 1, ls + _LEAD, ls + _LEAD)

        for b in range(_NBUF):
            scatter_wait(b)

    return emb


def kernel(x, input_embedding_table):
    nb, nl = x.shape
    idx4 = (
        x.T.reshape(nl // _NBUF, _NBUF, _NW, _CHUNK).transpose(0, 2, 1, 3)
    )
    out = _make_emb_kernel(nl, nb)(idx4, input_embedding_table)
    return out.reshape(nb, nl, _EMBED)


# speedup vs baseline: 3.2225x; 1.0002x over previous
import functools

import jax
import jax.numpy as jnp
from jax import lax
from jax.experimental import pallas as pl
from jax.experimental.pallas import tpu as pltpu
from jax.experimental.pallas import tpu_sc as plsc

_EMBED = 64
_LANES = 16
_NC = 2
_NS = 16
_NW = _NC * _NS
_CHUNK = 128
_NBUF = 8
_LEAD = 6


@functools.lru_cache(maxsize=None)
def _make_idx_reorder(nb: int, nl: int):
    def body(x_ref, o_ref):
        o_ref[...] = (
            x_ref[...]
            .reshape(_NW, _CHUNK, nl)
            .transpose(0, 2, 1)
            .reshape(nb * nl // _CHUNK, _CHUNK)
        )

    return pl.pallas_call(
        body,
        out_shape=jax.ShapeDtypeStruct((nb * nl // _CHUNK, _CHUNK), jnp.int32),
    )


@functools.lru_cache(maxsize=None)
def _make_emb_kernel(nl: int, nb: int):
    npw = nl * _CHUNK
    assert nb == _NW * _CHUNK and nl % _NBUF == 0 and nl // _NBUF >= 3
    mesh = plsc.VectorSubcoreMesh(core_axis_name="c", subcore_axis_name="s")

    @functools.partial(
        pl.kernel,
        out_type=jax.ShapeDtypeStruct((nb, nl * _EMBED), jnp.float32),
        mesh=mesh,
        scratch_types=[
            pltpu.VMEM((npw,), jnp.int32),
            pltpu.VMEM((_NBUF, _CHUNK, _EMBED), jnp.float32),
            pltpu.SemaphoreType.DMA((_NBUF,)),
            pltpu.SemaphoreType.DMA((_NBUF,)),
        ],
        compiler_params=pltpu.CompilerParams(use_tc_tiling_on_sc=False),
    )
    def emb(idx_hbm, table_hbm, out_hbm, idx_v, rows_v, gsem, ssem):
        wid = lax.axis_index("s") * _NC + lax.axis_index("c")
        col = wid * _CHUNK
        pltpu.sync_copy(idx_hbm.at[pl.ds(wid * npw, npw)], idx_v)

        def gather_issue(l, b):
            pltpu.async_copy(
                table_hbm.at[idx_v.at[pl.ds(l * _CHUNK, _CHUNK)]],
                rows_v.at[b],
                gsem.at[b],
            )

        def gather_wait(b):
            pltpu.make_async_copy(
                table_hbm.at[pl.ds(0, _CHUNK)], rows_v.at[b], gsem.at[b]
            ).wait()

        def scatter_issue(l, b):
            pltpu.async_copy(
                rows_v.at[b],
                out_hbm.at[pl.ds(col, _CHUNK), pl.ds(l * _EMBED, _EMBED)],
                ssem.at[b],
            )

        def scatter_wait(b):
            pltpu.make_async_copy(
                rows_v.at[b],
                out_hbm.at[pl.ds(0, _CHUNK), pl.ds(0, _EMBED)],
                ssem.at[b],
            ).wait()

        def scale(b):
            @pl.loop(0, _CHUNK, unroll=8)
            def _(i):
                for j in range(_EMBED // _LANES):
                    sl = pl.ds(j * _LANES, _LANES)
                    rows_v[b, i, sl] = rows_v[b, i, sl] * 8.0

        for ls in range(_LEAD):
            gather_issue(ls, ls)

        for ls in range(_NBUF):
            gather_wait(ls)
            scale(ls)
            scatter_issue(ls, ls)
            if ls >= 2:
                scatter_wait((ls - 2) % _NBUF)
            gather_issue(ls + _LEAD, (ls + _LEAD) % _NBUF)

        @pl.loop(1, nl // _NBUF - 1)
        def _(g):
            l0 = g * _NBUF
            for ls in range(_NBUF):
                gather_wait(ls)
                scale(ls)
                scatter_issue(l0 + ls, ls)
                scatter_wait((ls + _LEAD) % _NBUF)
                gather_issue(l0 + ls + _LEAD, (ls + _LEAD) % _NBUF)

        for ls in range(_NBUF):
            gather_wait(ls)
            scale(ls)
            scatter_issue(nl - _NBUF + ls, ls)
            if ls + _LEAD < _NBUF:
                scatter_wait(ls + _LEAD)
                gather_issue(nl - _NBUF + ls + _LEAD, ls + _LEAD)

        for b in range(_NBUF):
            scatter_wait(b)

    return emb


def kernel(x, input_embedding_table):
    nb, nl = x.shape
    idxf = _make_idx_reorder(nb, nl)(x).reshape(nb * nl)
    out = _make_emb_kernel(nl, nb)(idxf, input_embedding_table)
    return out.reshape(nb, nl, _EMBED)
